# baseline (speedup 1.0000x reference)
import functools

import jax
import jax.numpy as jnp
from jax import lax
from jax.experimental import pallas as pl
from jax.experimental.pallas import tpu as pltpu
from jax.experimental.pallas import tpu_sc as plsc

N_NODES = 10000
N_EDGES = 320000
D = 128
L = 16

NC = 2
NS = 16
NW = NC * NS
CH = 128
NCHUNK = 80
GRP = 8
NGRP = NCHUNK // GRP
E_PAD = NW * NCHUNK * CH
DUMP = N_NODES
ACC_ROWS = 10112
ZROWS = ACC_ROWS // NS
DEG_ROWS = 10112
N_DUMP = ACC_ROWS - N_NODES


def _sc_aggregate(nodes, send_p, recv_p):
    mesh = plsc.VectorSubcoreMesh(core_axis_name="c", subcore_axis_name="s")

    @functools.partial(
        pl.kernel,
        mesh=mesh,
        out_type=(
            jax.ShapeDtypeStruct((NC, ACC_ROWS, D), jnp.float32),
            jax.ShapeDtypeStruct((NW, DEG_ROWS), jnp.float32),
        ),
        scratch_types=[
            pltpu.VMEM_SHARED((ACC_ROWS, D), jnp.float32),
            pltpu.VMEM((2, GRP, CH), jnp.int32),
            pltpu.VMEM((2, GRP, CH), jnp.int32),
            pltpu.VMEM((2, CH, D), jnp.float32),
            pltpu.VMEM((DEG_ROWS,), jnp.float32),
            pltpu.SemaphoreType.DMA,
            pltpu.SemaphoreType.DMA,
            pltpu.SemaphoreType.DMA,
            pltpu.SemaphoreType.DMA,
        ],
        compiler_params=pltpu.CompilerParams(needs_layout_passes=False),
    )
    def k(nodes_h, send_h, recv_h, sum_h, deg_h, acc_sh,
          send_v, recv_v, rows_v, deg_v, sem_r0, sem_r1, sem_idx, sem_sc):
        c = lax.axis_index("c")
        s = lax.axis_index("s")
        wid = c * NS + s

        zv = jnp.zeros((L,), jnp.float32)
        ov = jnp.ones((L,), jnp.float32)

        @pl.loop(0, CH)
        def _(i):
            for j in range(D // L):
                rows_v[0, i, pl.ds(j * L, L)] = zv

        @pl.loop(0, DEG_ROWS // L)
        def _(i):
            deg_v[pl.ds(i * L, L)] = zv

        base = s * ZROWS
        for o in range(0, ZROWS, CH):
            n = min(CH, ZROWS - o)
            pltpu.sync_copy(rows_v.at[0].at[pl.ds(0, n)],
                            acc_sh.at[pl.ds(base + o, n)])

        pltpu.sync_copy(send_h.at[wid].at[pl.ds(0, GRP)], send_v.at[0])
        pltpu.sync_copy(recv_h.at[wid].at[pl.ds(0, GRP)], recv_v.at[0])
        pltpu.async_copy(
            nodes_h.at[send_v.at[0].at[0]], rows_v.at[0], sem_r0)

        plsc.subcore_barrier()

        sems = (sem_r0, sem_r1)

        @pl.loop(0, NGRP)
        def _(g):
            slot_g = lax.rem(g, 2)
            slot_n = 1 - slot_g

            @pl.when(g < NGRP - 1)
            def _():
                off = (g + 1) * GRP
                pltpu.async_copy(send_h.at[wid].at[pl.ds(off, GRP)],
                                 send_v.at[slot_n], sem_idx)
                pltpu.async_copy(recv_h.at[wid].at[pl.ds(off, GRP)],
                                 recv_v.at[slot_n], sem_idx)

            for jj in range(GRP):
                rs = jj % 2
                pltpu.make_async_copy(
                    nodes_h.at[send_v.at[slot_g].at[jj].at[pl.ds(0, 64)]],
                    rows_v.at[rs].at[pl.ds(0, 64)], sems[rs]).wait()
                pltpu.make_async_copy(
                    nodes_h.at[send_v.at[slot_g].at[jj].at[pl.ds(64, 64)]],
                    rows_v.at[rs].at[pl.ds(64, 64)], sems[rs]).wait()
                if jj < GRP - 1:
                    pltpu.async_copy(
                        nodes_h.at[send_v.at[slot_g].at[jj + 1].at[pl.ds(0, 64)]],
                        rows_v.at[1 - rs].at[pl.ds(0, 64)], sems[1 - rs])
                    pltpu.async_copy(
                        nodes_h.at[send_v.at[slot_g].at[jj + 1].at[pl.ds(64, 64)]],
                        rows_v.at[1 - rs].at[pl.ds(64, 64)], sems[1 - rs])
                else:
                    @pl.when(g < NGRP - 1)
                    def _():
                        pltpu.make_async_copy(
                            send_h.at[wid].at[pl.ds(0, GRP)],
                            send_v.at[slot_n], sem_idx).wait()
                        pltpu.make_async_copy(
                            recv_h.at[wid].at[pl.ds(0, GRP)],
                            recv_v.at[slot_n], sem_idx).wait()
                        pltpu.async_copy(
                            nodes_h.at[send_v.at[slot_n].at[0]],
                            rows_v.at[0], sem_r0)
                for kk in range(CH // L):
                    rv = recv_v[slot_g, jj, pl.ds(kk * L, L)]
                    plsc.addupdate_scatter(deg_v, [rv], ov)

        plsc.subcore_barrier()

        pltpu.sync_copy(acc_sh.at[pl.ds(base, ZROWS)],
                        sum_h.at[c].at[pl.ds(base, ZROWS)])
        pltpu.sync_copy(deg_v, deg_h.at[wid])

    return k(nodes, send_p, recv_p)


def _tc_finish_body(ps_ref, pd_ref, w_ref, b_ref, out_ref):
    agg = ps_ref[0] + ps_ref[1]
    deg = jnp.maximum(jnp.sum(pd_ref[...], axis=0), 1.0)
    x = agg / deg[:, None]
    out_ref[...] = (
        jnp.dot(x, w_ref[...], preferred_element_type=jnp.float32)
        + b_ref[...]
    )


def _tc_finish(part_sum, part_deg, W, b):
    R = 2048
    grid = (pl.cdiv(N_NODES, R),)
    return pl.pallas_call(
        _tc_finish_body,
        grid=grid,
        in_specs=[
            pl.BlockSpec((NC, R, D), lambda i: (0, i, 0)),
            pl.BlockSpec((NW, R), lambda i: (0, i)),
            pl.BlockSpec((D, D), lambda i: (0, 0)),
            pl.BlockSpec((1, D), lambda i: (0, 0)),
        ],
        out_specs=pl.BlockSpec((R, D), lambda i: (i, 0)),
        out_shape=jax.ShapeDtypeStruct((N_NODES, D), jnp.float32),
    )(part_sum, part_deg, W, b.reshape(1, D))


@jax.jit
def kernel(nodes, senders, receivers, W, b):
    senders = senders.astype(jnp.int32)
    receivers = receivers.astype(jnp.int32)
    pad = E_PAD - N_EDGES
    pad_iota = jnp.arange(pad, dtype=jnp.int32)
    send_p = jnp.concatenate(
        [senders, pad_iota % 128]).reshape(NW, NCHUNK, CH)
    recv_p = jnp.concatenate(
        [receivers, DUMP + pad_iota % N_DUMP]).reshape(NW, NCHUNK, CH)
    part_sum, part_deg = _sc_aggregate(nodes, send_p, recv_p)
    return _tc_finish(part_sum, part_deg, W, b)


# speedup vs baseline: 1.1733x; 1.1733x over previous
import functools

import jax
import jax.numpy as jnp
from jax import lax
from jax.experimental import pallas as pl
from jax.experimental.pallas import tpu as pltpu
from jax.experimental.pallas import tpu_sc as plsc

N_NODES = 10000
N_EDGES = 320000
D = 128
L = 16

NC = 2
NS = 16
NW = NC * NS
CH = 80
NCHUNK = 128
GRP = 8
NGRP = NCHUNK // GRP
E_PAD = NW * NCHUNK * CH
DUMP = N_NODES
ACC_ROWS = 10112
ZROWS = ACC_ROWS // NS
DEG_ROWS = 10112
N_DUMP = ACC_ROWS - N_NODES


def _sc_aggregate(nodes, send_p, recv_p):
    mesh = plsc.VectorSubcoreMesh(core_axis_name="c", subcore_axis_name="s")

    @functools.partial(
        pl.kernel,
        mesh=mesh,
        out_type=(
            jax.ShapeDtypeStruct((NC, ACC_ROWS, D), jnp.float32),
            jax.ShapeDtypeStruct((NW, DEG_ROWS), jnp.float32),
        ),
        scratch_types=[
            pltpu.VMEM_SHARED((ACC_ROWS, D), jnp.float32),
            pltpu.VMEM((2, GRP, CH), jnp.int32),
            pltpu.VMEM((2, GRP, CH), jnp.int32),
            pltpu.VMEM((3, CH, D), jnp.float32),
            pltpu.VMEM((DEG_ROWS,), jnp.float32),
            pltpu.SemaphoreType.DMA,
            pltpu.SemaphoreType.DMA,
            pltpu.SemaphoreType.DMA,
        ],
        compiler_params=pltpu.CompilerParams(needs_layout_passes=False),
    )
    def k(nodes_h, send_h, recv_h, sum_h, deg_h, acc_sh,
          send_v, recv_v, rows_v, deg_v, sem_r, sem_idx, sem_sc):
        c = lax.axis_index("c")
        s = lax.axis_index("s")
        wid = c * NS + s

        zv = jnp.zeros((L,), jnp.float32)
        ov = jnp.ones((L,), jnp.float32)

        @pl.loop(0, CH)
        def _(i):
            for j in range(D // L):
                rows_v[0, i, pl.ds(j * L, L)] = zv

        @pl.loop(0, DEG_ROWS // L)
        def _(i):
            deg_v[pl.ds(i * L, L)] = zv

        base = s * ZROWS
        for o in range(0, ZROWS, CH):
            n = min(CH, ZROWS - o)
            pltpu.sync_copy(rows_v.at[0].at[pl.ds(0, n)],
                            acc_sh.at[pl.ds(base + o, n)])

        pltpu.sync_copy(send_h.at[wid].at[pl.ds(0, GRP)], send_v.at[0])
        pltpu.sync_copy(recv_h.at[wid].at[pl.ds(0, GRP)], recv_v.at[0])
        pltpu.async_copy(
            nodes_h.at[send_v.at[0].at[0]], rows_v.at[0], sem_r)
        pltpu.async_copy(
            nodes_h.at[send_v.at[0].at[1]], rows_v.at[1], sem_r)

        plsc.subcore_barrier()

        @pl.loop(0, NGRP)
        def _(g):
            slot_g = lax.rem(g, 2)
            slot_n = 1 - slot_g

            @pl.when(g < NGRP - 1)
            def _():
                off = (g + 1) * GRP
                pltpu.async_copy(send_h.at[wid].at[pl.ds(off, GRP)],
                                 send_v.at[slot_n], sem_idx)
                pltpu.async_copy(recv_h.at[wid].at[pl.ds(off, GRP)],
                                 recv_v.at[slot_n], sem_idx)

            for jj in range(GRP):
                rs = lax.rem(g * GRP + jj, 3)
                ns = lax.rem(g * GRP + jj + 2, 3)
                pltpu.make_async_copy(
                    nodes_h.at[send_v.at[slot_g].at[jj]],
                    rows_v.at[rs], sem_r).wait()
                if jj > 0:
                    pltpu.make_async_copy(
                        rows_v.at[0],
                        acc_sh.at[recv_v.at[0].at[0]],
                        sem_sc).wait()
                else:
                    @pl.when(g > 0)
                    def _():
                        pltpu.make_async_copy(
                            rows_v.at[0],
                            acc_sh.at[recv_v.at[0].at[0]],
                            sem_sc).wait()
                if jj < GRP - 2:
                    pltpu.async_copy(
                        nodes_h.at[send_v.at[slot_g].at[jj + 2]],
                        rows_v.at[ns], sem_r)
                elif jj == GRP - 2:
                    @pl.when(g < NGRP - 1)
                    def _():
                        pltpu.make_async_copy(
                            send_h.at[wid].at[pl.ds(0, GRP)],
                            send_v.at[slot_n], sem_idx).wait()
                        pltpu.make_async_copy(
                            recv_h.at[wid].at[pl.ds(0, GRP)],
                            recv_v.at[slot_n], sem_idx).wait()
                        pltpu.async_copy(
                            nodes_h.at[send_v.at[slot_n].at[0]],
                            rows_v.at[ns], sem_r)
                else:
                    @pl.when(g < NGRP - 1)
                    def _():
                        pltpu.async_copy(
                            nodes_h.at[send_v.at[slot_n].at[1]],
                            rows_v.at[ns], sem_r)
                pltpu.async_copy(rows_v.at[rs],
                                 acc_sh.at[recv_v.at[slot_g].at[jj]],
                                 sem_sc, add=True)
                for kk in range(CH // L):
                    rv = recv_v[slot_g, jj, pl.ds(kk * L, L)]
                    plsc.addupdate_scatter(deg_v, [rv], ov)

        pltpu.make_async_copy(
            rows_v.at[0], acc_sh.at[recv_v.at[0].at[0]], sem_sc).wait()

        plsc.subcore_barrier()

        pltpu.sync_copy(acc_sh.at[pl.ds(base, ZROWS)],
                        sum_h.at[c].at[pl.ds(base, ZROWS)])
        pltpu.sync_copy(deg_v, deg_h.at[wid])

    return k(nodes, send_p, recv_p)


def _tc_finish_body(ps_ref, pd_ref, w_ref, b_ref, out_ref):
    agg = ps_ref[0] + ps_ref[1]
    deg = jnp.maximum(jnp.sum(pd_ref[...], axis=0), 1.0)
    x = agg / deg[:, None]
    out_ref[...] = (
        jnp.dot(x, w_ref[...], preferred_element_type=jnp.float32)
        + b_ref[...]
    )


def _tc_finish(part_sum, part_deg, W, b):
    R = 2048
    grid = (pl.cdiv(N_NODES, R),)
    return pl.pallas_call(
        _tc_finish_body,
        grid=grid,
        in_specs=[
            pl.BlockSpec((NC, R, D), lambda i: (0, i, 0)),
            pl.BlockSpec((NW, R), lambda i: (0, i)),
            pl.BlockSpec((D, D), lambda i: (0, 0)),
            pl.BlockSpec((1, D), lambda i: (0, 0)),
        ],
        out_specs=pl.BlockSpec((R, D), lambda i: (i, 0)),
        out_shape=jax.ShapeDtypeStruct((N_NODES, D), jnp.float32),
    )(part_sum, part_deg, W, b.reshape(1, D))


@jax.jit
def kernel(nodes, senders, receivers, W, b):
    senders = senders.astype(jnp.int32)
    receivers = receivers.astype(jnp.int32)
    pad = E_PAD - N_EDGES
    pad_iota = jnp.arange(pad, dtype=jnp.int32)
    send_p = jnp.concatenate(
        [senders, pad_iota % 128]).reshape(NW, NCHUNK, CH)
    recv_p = jnp.concatenate(
        [receivers, DUMP + pad_iota % N_DUMP]).reshape(NW, NCHUNK, CH)
    part_sum, part_deg = _sc_aggregate(nodes, send_p, recv_p)
    return _tc_finish(part_sum, part_deg, W, b)


# speedup vs baseline: 1.2431x; 1.0594x over previous
import functools

import jax
import jax.numpy as jnp
from jax import lax
from jax.experimental import pallas as pl
from jax.experimental.pallas import tpu as pltpu
from jax.experimental.pallas import tpu_sc as plsc

N_NODES = 10000
N_EDGES = 320000
D = 128
L = 16

NC = 2
NS = 16
NW = NC * NS
CH = 64
NCHUNK = 160
GRP = 8
NGRP = NCHUNK // GRP
E_PAD = NW * NCHUNK * CH
DUMP = N_NODES
ACC_ROWS = 10112
ZROWS = ACC_ROWS // NS
DEG_ROWS = 10112
N_DUMP = ACC_ROWS - N_NODES


def _sc_aggregate(nodes, send_p, recv_p):
    mesh = plsc.VectorSubcoreMesh(core_axis_name="c", subcore_axis_name="s")

    @functools.partial(
        pl.kernel,
        mesh=mesh,
        out_type=(
            jax.ShapeDtypeStruct((NC, ACC_ROWS, D), jnp.float32),
            jax.ShapeDtypeStruct((NW, DEG_ROWS), jnp.float32),
        ),
        scratch_types=[
            pltpu.VMEM_SHARED((ACC_ROWS, D), jnp.float32),
            pltpu.VMEM((2, GRP, CH), jnp.int32),
            pltpu.VMEM((2, GRP, CH), jnp.int32),
            pltpu.VMEM((4, CH, D), jnp.float32),
            pltpu.VMEM((DEG_ROWS,), jnp.float32),
            pltpu.SemaphoreType.DMA,
            pltpu.SemaphoreType.DMA,
            pltpu.SemaphoreType.DMA,
            pltpu.SemaphoreType.DMA,
            pltpu.SemaphoreType.DMA,
            pltpu.SemaphoreType.DMA,
        ],
        compiler_params=pltpu.CompilerParams(needs_layout_passes=False),
    )
    def k(nodes_h, send_h, recv_h, sum_h, deg_h, acc_sh,
          send_v, recv_v, rows_v, deg_v,
          sem_r0, sem_r1, sem_r2, sem_r3, sem_idx, sem_sc):
        c = lax.axis_index("c")
        s = lax.axis_index("s")
        wid = c * NS + s
        sems = (sem_r0, sem_r1, sem_r2, sem_r3)

        zv = jnp.zeros((L,), jnp.float32)
        ov = jnp.ones((L,), jnp.float32)

        @pl.loop(0, CH)
        def _(i):
            for j in range(D // L):
                rows_v[0, i, pl.ds(j * L, L)] = zv

        @pl.loop(0, DEG_ROWS // L)
        def _(i):
            deg_v[pl.ds(i * L, L)] = zv

        base = s * ZROWS
        for o in range(0, ZROWS, CH):
            n = min(CH, ZROWS - o)
            pltpu.sync_copy(rows_v.at[0].at[pl.ds(0, n)],
                            acc_sh.at[pl.ds(base + o, n)])

        pltpu.sync_copy(send_h.at[wid].at[pl.ds(0, GRP)], send_v.at[0])
        pltpu.sync_copy(recv_h.at[wid].at[pl.ds(0, GRP)], recv_v.at[0])
        pltpu.async_copy(
            nodes_h.at[send_v.at[0].at[0]], rows_v.at[0], sem_r0)
        pltpu.async_copy(
            nodes_h.at[send_v.at[0].at[1]], rows_v.at[1], sem_r1)
        pltpu.async_copy(
            nodes_h.at[send_v.at[0].at[2]], rows_v.at[2], sem_r2)

        plsc.subcore_barrier()

        @pl.loop(0, NGRP)
        def _(g):
            slot_g = lax.rem(g, 2)
            slot_n = 1 - slot_g

            @pl.when(g < NGRP - 1)
            def _():
                off = (g + 1) * GRP
                pltpu.async_copy(send_h.at[wid].at[pl.ds(off, GRP)],
                                 send_v.at[slot_n], sem_idx)
                pltpu.async_copy(recv_h.at[wid].at[pl.ds(off, GRP)],
                                 recv_v.at[slot_n], sem_idx)

            for jj in range(GRP):
                rs = jj % 4
                ns = (jj + 3) % 4
                pltpu.make_async_copy(
                    nodes_h.at[send_v.at[slot_g].at[jj]],
                    rows_v.at[rs], sems[rs]).wait()
                if jj > 0:
                    pltpu.make_async_copy(
                        rows_v.at[0],
                        acc_sh.at[recv_v.at[0].at[0]],
                        sem_sc).wait()
                else:
                    @pl.when(g > 0)
                    def _():
                        pltpu.make_async_copy(
                            rows_v.at[0],
                            acc_sh.at[recv_v.at[0].at[0]],
                            sem_sc).wait()
                if jj < GRP - 3:
                    pltpu.async_copy(
                        nodes_h.at[send_v.at[slot_g].at[jj + 3]],
                        rows_v.at[ns], sems[ns])
                elif jj == GRP - 3:
                    @pl.when(g < NGRP - 1)
                    def _():
                        pltpu.make_async_copy(
                            send_h.at[wid].at[pl.ds(0, GRP)],
                            send_v.at[slot_n], sem_idx).wait()
                        pltpu.make_async_copy(
                            recv_h.at[wid].at[pl.ds(0, GRP)],
                            recv_v.at[slot_n], sem_idx).wait()
                        pltpu.async_copy(
                            nodes_h.at[send_v.at[slot_n].at[0]],
                            rows_v.at[ns], sems[ns])
                else:
                    @pl.when(g < NGRP - 1)
                    def _():
                        pltpu.async_copy(
                            nodes_h.at[send_v.at[slot_n].at[jj - (GRP - 3)]],
                            rows_v.at[ns], sems[ns])
                pltpu.async_copy(rows_v.at[rs],
                                 acc_sh.at[recv_v.at[slot_g].at[jj]],
                                 sem_sc, add=True)
                for kk in range(CH // L):
                    rv = recv_v[slot_g, jj, pl.ds(kk * L, L)]
                    plsc.addupdate_scatter(deg_v, [rv], ov)

        pltpu.make_async_copy(
            rows_v.at[0], acc_sh.at[recv_v.at[0].at[0]], sem_sc).wait()

        plsc.subcore_barrier()

        pltpu.sync_copy(acc_sh.at[pl.ds(base, ZROWS)],
                        sum_h.at[c].at[pl.ds(base, ZROWS)])
        pltpu.sync_copy(deg_v, deg_h.at[wid])

    return k(nodes, send_p, recv_p)


def _tc_finish_body(ps_ref, pd_ref, w_ref, b_ref, out_ref):
    agg = ps_ref[0] + ps_ref[1]
    deg = jnp.maximum(jnp.sum(pd_ref[...], axis=0), 1.0)
    x = agg / deg[:, None]
    out_ref[...] = (
        jnp.dot(x, w_ref[...], preferred_element_type=jnp.float32)
        + b_ref[...]
    )


def _tc_finish(part_sum, part_deg, W, b):
    R = 2048
    grid = (pl.cdiv(N_NODES, R),)
    return pl.pallas_call(
        _tc_finish_body,
        grid=grid,
        in_specs=[
            pl.BlockSpec((NC, R, D), lambda i: (0, i, 0)),
            pl.BlockSpec((NW, R), lambda i: (0, i)),
            pl.BlockSpec((D, D), lambda i: (0, 0)),
            pl.BlockSpec((1, D), lambda i: (0, 0)),
        ],
        out_specs=pl.BlockSpec((R, D), lambda i: (i, 0)),
        out_shape=jax.ShapeDtypeStruct((N_NODES, D), jnp.float32),
    )(part_sum, part_deg, W, b.reshape(1, D))


@jax.jit
def kernel(nodes, senders, receivers, W, b):
    senders = senders.astype(jnp.int32)
    receivers = receivers.astype(jnp.int32)
    pad = E_PAD - N_EDGES
    pad_iota = jnp.arange(pad, dtype=jnp.int32)
    send_p = jnp.concatenate(
        [senders, pad_iota % 128]).reshape(NW, NCHUNK, CH)
    recv_p = jnp.concatenate(
        [receivers, DUMP + pad_iota % N_DUMP]).reshape(NW, NCHUNK, CH)
    part_sum, part_deg = _sc_aggregate(nodes, send_p, recv_p)
    return _tc_finish(part_sum, part_deg, W, b)


# speedup vs baseline: 1.2799x; 1.0296x over previous
import functools

import jax
import jax.numpy as jnp
from jax import lax
from jax.experimental import pallas as pl
from jax.experimental.pallas import tpu as pltpu
from jax.experimental.pallas import tpu_sc as plsc

N_NODES = 10000
N_EDGES = 320000
D = 128
L = 16

NC = 2
NS = 16
NW = NC * NS
CH = 64
NCHUNK = 160
GRP = 8
NGRP = NCHUNK // GRP
E_PAD = NW * NCHUNK * CH
DUMP = N_NODES
ACC_ROWS = 10112
ZROWS = ACC_ROWS // NS
DEG_ROWS = 10112
N_DUMP = ACC_ROWS - N_NODES


def _sc_aggregate(nodes, send_p, recv_p):
    mesh = plsc.VectorSubcoreMesh(core_axis_name="c", subcore_axis_name="s")

    @functools.partial(
        pl.kernel,
        mesh=mesh,
        out_type=(
            jax.ShapeDtypeStruct((NC, ACC_ROWS, D), jnp.float32),
            jax.ShapeDtypeStruct((NW, DEG_ROWS), jnp.float32),
        ),
        scratch_types=[
            pltpu.VMEM_SHARED((ACC_ROWS, D), jnp.float32),
            pltpu.VMEM((2, GRP, CH), jnp.int32),
            pltpu.VMEM((2, GRP, CH), jnp.int32),
            pltpu.VMEM((4, CH, D), jnp.float32),
            pltpu.VMEM((DEG_ROWS,), jnp.float32),
            pltpu.SemaphoreType.DMA,
            pltpu.SemaphoreType.DMA,
            pltpu.SemaphoreType.DMA,
            pltpu.SemaphoreType.DMA,
            pltpu.SemaphoreType.DMA,
            pltpu.SemaphoreType.DMA,
        ],
        compiler_params=pltpu.CompilerParams(needs_layout_passes=False),
    )
    def k(nodes_h, send_h, recv_h, sum_h, deg_h, acc_sh,
          send_v, recv_v, rows_v, deg_v,
          sem_r0, sem_r1, sem_r2, sem_r3, sem_idx, sem_sc):
        c = lax.axis_index("c")
        s = lax.axis_index("s")
        wid = c * NS + s
        sems = (sem_r0, sem_r1, sem_r2, sem_r3)

        zv = jnp.zeros((L,), jnp.float32)
        ov = jnp.ones((L,), jnp.float32)

        pltpu.sync_copy(send_h.at[wid].at[pl.ds(0, GRP)], send_v.at[0])
        pltpu.sync_copy(recv_h.at[wid].at[pl.ds(0, GRP)], recv_v.at[0])
        pltpu.async_copy(
            nodes_h.at[send_v.at[0].at[0]], rows_v.at[0], sem_r0)
        pltpu.async_copy(
            nodes_h.at[send_v.at[0].at[1]], rows_v.at[1], sem_r1)
        pltpu.async_copy(
            nodes_h.at[send_v.at[0].at[2]], rows_v.at[2], sem_r2)

        @pl.loop(0, CH, unroll=8)
        def _(i):
            for j in range(D // L):
                rows_v[3, i, pl.ds(j * L, L)] = zv

        @pl.loop(0, DEG_ROWS // L, unroll=8)
        def _(i):
            deg_v[pl.ds(i * L, L)] = zv

        base = s * ZROWS
        for o in range(0, ZROWS, CH):
            n = min(CH, ZROWS - o)
            pltpu.sync_copy(rows_v.at[3].at[pl.ds(0, n)],
                            acc_sh.at[pl.ds(base + o, n)])

        plsc.subcore_barrier()

        @pl.loop(0, NGRP)
        def _(g):
            slot_g = lax.rem(g, 2)
            slot_n = 1 - slot_g

            @pl.when(g < NGRP - 1)
            def _():
                off = (g + 1) * GRP
                pltpu.async_copy(send_h.at[wid].at[pl.ds(off, GRP)],
                                 send_v.at[slot_n], sem_idx)
                pltpu.async_copy(recv_h.at[wid].at[pl.ds(off, GRP)],
                                 recv_v.at[slot_n], sem_idx)

            for jj in range(GRP):
                rs = jj % 4
                ns = (jj + 3) % 4
                pltpu.make_async_copy(
                    nodes_h.at[send_v.at[slot_g].at[jj]],
                    rows_v.at[rs], sems[rs]).wait()
                if jj > 0:
                    pltpu.make_async_copy(
                        rows_v.at[0],
                        acc_sh.at[recv_v.at[0].at[0]],
                        sem_sc).wait()
                else:
                    @pl.when(g > 0)
                    def _():
                        pltpu.make_async_copy(
                            rows_v.at[0],
                            acc_sh.at[recv_v.at[0].at[0]],
                            sem_sc).wait()
                if jj < GRP - 3:
                    pltpu.async_copy(
                        nodes_h.at[send_v.at[slot_g].at[jj + 3]],
                        rows_v.at[ns], sems[ns])
                elif jj == GRP - 3:
                    @pl.when(g < NGRP - 1)
                    def _():
                        pltpu.make_async_copy(
                            send_h.at[wid].at[pl.ds(0, GRP)],
                            send_v.at[slot_n], sem_idx).wait()
                        pltpu.make_async_copy(
                            recv_h.at[wid].at[pl.ds(0, GRP)],
                            recv_v.at[slot_n], sem_idx).wait()
                        pltpu.async_copy(
                            nodes_h.at[send_v.at[slot_n].at[0]],
                            rows_v.at[ns], sems[ns])
                else:
                    @pl.when(g < NGRP - 1)
                    def _():
                        pltpu.async_copy(
                            nodes_h.at[send_v.at[slot_n].at[jj - (GRP - 3)]],
                            rows_v.at[ns], sems[ns])
                pltpu.async_copy(rows_v.at[rs],
                                 acc_sh.at[recv_v.at[slot_g].at[jj]],
                                 sem_sc, add=True)
                for kk in range(CH // L):
                    rv = recv_v[slot_g, jj, pl.ds(kk * L, L)]
                    plsc.addupdate_scatter(deg_v, [rv], ov)

        pltpu.make_async_copy(
            rows_v.at[0], acc_sh.at[recv_v.at[0].at[0]], sem_sc).wait()

        plsc.subcore_barrier()

        pltpu.sync_copy(acc_sh.at[pl.ds(base, ZROWS)],
                        sum_h.at[c].at[pl.ds(base, ZROWS)])
        pltpu.sync_copy(deg_v, deg_h.at[wid])

    return k(nodes, send_p, recv_p)


def _tc_finish_body(ps_ref, pd_ref, w_ref, b_ref, out_ref):
    agg = ps_ref[0] + ps_ref[1]
    deg = jnp.maximum(jnp.sum(pd_ref[...], axis=0), 1.0)
    x = agg / deg[:, None]
    out_ref[...] = (
        jnp.dot(x, w_ref[...], preferred_element_type=jnp.float32)
        + b_ref[...]
    )


def _tc_finish(part_sum, part_deg, W, b):
    R = 2048
    grid = (pl.cdiv(N_NODES, R),)
    return pl.pallas_call(
        _tc_finish_body,
        grid=grid,
        in_specs=[
            pl.BlockSpec((NC, R, D), lambda i: (0, i, 0)),
            pl.BlockSpec((NW, R), lambda i: (0, i)),
            pl.BlockSpec((D, D), lambda i: (0, 0)),
            pl.BlockSpec((1, D), lambda i: (0, 0)),
        ],
        out_specs=pl.BlockSpec((R, D), lambda i: (i, 0)),
        out_shape=jax.ShapeDtypeStruct((N_NODES, D), jnp.float32),
    )(part_sum, part_deg, W, b.reshape(1, D))


@jax.jit
def kernel(nodes, senders, receivers, W, b):
    senders = senders.astype(jnp.int32)
    receivers = receivers.astype(jnp.int32)
    pad = E_PAD - N_EDGES
    pad_iota = jnp.arange(pad, dtype=jnp.int32)
    send_p = jnp.concatenate(
        [senders, pad_iota % 128]).reshape(NW, NCHUNK, CH)
    recv_p = jnp.concatenate(
        [receivers, DUMP + pad_iota % N_DUMP]).reshape(NW, NCHUNK, CH)
    part_sum, part_deg = _sc_aggregate(nodes, send_p, recv_p)
    return _tc_finish(part_sum, part_deg, W, b)


# speedup vs baseline: 1.2879x; 1.0063x over previous
import functools

import jax
import jax.numpy as jnp
from jax import lax
from jax.experimental import pallas as pl
from jax.experimental.pallas import tpu as pltpu
from jax.experimental.pallas import tpu_sc as plsc

N_NODES = 10000
N_EDGES = 320000
D = 128
L = 16

NC = 2
NS = 16
NW = NC * NS
CH = 64
NCHUNK = 160
GRP = 8
NGRP = NCHUNK // GRP
E_PAD = NW * NCHUNK * CH
DUMP = N_NODES
ACC_ROWS = 10112
ZROWS = ACC_ROWS // NS
DEG_ROWS = 10112
N_DUMP = ACC_ROWS - N_NODES


def _sc_aggregate(nodes, send_p, recv_p):
    mesh = plsc.VectorSubcoreMesh(core_axis_name="c", subcore_axis_name="s")

    @functools.partial(
        pl.kernel,
        mesh=mesh,
        out_type=(
            jax.ShapeDtypeStruct((NC, ACC_ROWS, D), jnp.float32),
            jax.ShapeDtypeStruct((NW, DEG_ROWS), jnp.float32),
        ),
        scratch_types=[
            pltpu.VMEM_SHARED((ACC_ROWS, D), jnp.float32),
            pltpu.VMEM((2, GRP, CH), jnp.int32),
            pltpu.VMEM((2, GRP, CH), jnp.int32),
            pltpu.VMEM((4, CH, D), jnp.float32),
            pltpu.VMEM((DEG_ROWS,), jnp.float32),
            pltpu.SemaphoreType.DMA,
            pltpu.SemaphoreType.DMA,
            pltpu.SemaphoreType.DMA,
            pltpu.SemaphoreType.DMA,
            pltpu.SemaphoreType.DMA,
            pltpu.SemaphoreType.DMA,
        ],
        compiler_params=pltpu.CompilerParams(needs_layout_passes=False),
    )
    def k(nodes_h, send_h, recv_h, sum_h, deg_h, acc_sh,
          send_v, recv_v, rows_v, deg_v,
          sem_r0, sem_r1, sem_r2, sem_r3, sem_idx, sem_sc):
        c = lax.axis_index("c")
        s = lax.axis_index("s")
        wid = c * NS + s
        sems = (sem_r0, sem_r1, sem_r2, sem_r3)

        zv = jnp.zeros((L,), jnp.float32)
        ov = jnp.ones((L,), jnp.float32)

        pltpu.sync_copy(send_h.at[wid].at[pl.ds(0, GRP)], send_v.at[0])
        pltpu.sync_copy(recv_h.at[wid].at[pl.ds(0, GRP)], recv_v.at[0])
        pltpu.async_copy(
            nodes_h.at[send_v.at[0].at[0]], rows_v.at[0], sem_r0)
        pltpu.async_copy(
            nodes_h.at[send_v.at[0].at[1]], rows_v.at[1], sem_r1)
        pltpu.async_copy(
            nodes_h.at[send_v.at[0].at[2]], rows_v.at[2], sem_r2)

        @pl.loop(0, CH, unroll=8)
        def _(i):
            for j in range(D // L):
                rows_v[3, i, pl.ds(j * L, L)] = zv

        @pl.loop(0, DEG_ROWS // L, unroll=8)
        def _(i):
            deg_v[pl.ds(i * L, L)] = zv

        base = s * ZROWS
        for o in range(0, ZROWS, CH):
            n = min(CH, ZROWS - o)
            pltpu.sync_copy(rows_v.at[3].at[pl.ds(0, n)],
                            acc_sh.at[pl.ds(base + o, n)])

        plsc.subcore_barrier()

        @pl.loop(0, NGRP)
        def _(g):
            slot_g = lax.rem(g, 2)
            slot_n = 1 - slot_g

            @pl.when(g < NGRP - 1)
            def _():
                off = (g + 1) * GRP
                pltpu.async_copy(send_h.at[wid].at[pl.ds(off, GRP)],
                                 send_v.at[slot_n], sem_idx)
                pltpu.async_copy(recv_h.at[wid].at[pl.ds(off, GRP)],
                                 recv_v.at[slot_n], sem_idx)

            for jj in range(GRP):
                rs = jj % 4
                ns = (jj + 3) % 4
                pltpu.make_async_copy(
                    nodes_h.at[send_v.at[slot_g].at[jj]],
                    rows_v.at[rs], sems[rs]).wait()
                if jj > 0:
                    pltpu.make_async_copy(
                        rows_v.at[0],
                        acc_sh.at[recv_v.at[0].at[0]],
                        sem_sc).wait()
                else:
                    @pl.when(g > 0)
                    def _():
                        pltpu.make_async_copy(
                            rows_v.at[0],
                            acc_sh.at[recv_v.at[0].at[0]],
                            sem_sc).wait()
                if jj < GRP - 3:
                    pltpu.async_copy(
                        nodes_h.at[send_v.at[slot_g].at[jj + 3]],
                        rows_v.at[ns], sems[ns])
                elif jj == GRP - 3:
                    @pl.when(g < NGRP - 1)
                    def _():
                        pltpu.make_async_copy(
                            send_h.at[wid].at[pl.ds(0, GRP)],
                            send_v.at[slot_n], sem_idx).wait()
                        pltpu.make_async_copy(
                            recv_h.at[wid].at[pl.ds(0, GRP)],
                            recv_v.at[slot_n], sem_idx).wait()
                        pltpu.async_copy(
                            nodes_h.at[send_v.at[slot_n].at[0]],
                            rows_v.at[ns], sems[ns])
                else:
                    @pl.when(g < NGRP - 1)
                    def _():
                        pltpu.async_copy(
                            nodes_h.at[send_v.at[slot_n].at[jj - (GRP - 3)]],
                            rows_v.at[ns], sems[ns])
                pltpu.async_copy(rows_v.at[rs],
                                 acc_sh.at[recv_v.at[slot_g].at[jj]],
                                 sem_sc, add=True)
                for kk in range(CH // L):
                    rv = recv_v[slot_g, jj, pl.ds(kk * L, L)]
                    plsc.addupdate_scatter(deg_v, [rv], ov)

        pltpu.make_async_copy(
            rows_v.at[0], acc_sh.at[recv_v.at[0].at[0]], sem_sc).wait()

        plsc.subcore_barrier()

        pltpu.async_copy(acc_sh.at[pl.ds(base, ZROWS)],
                         sum_h.at[c].at[pl.ds(base, ZROWS)], sem_sc)
        pltpu.async_copy(deg_v, deg_h.at[wid], sem_r0)
        pltpu.make_async_copy(acc_sh.at[pl.ds(base, ZROWS)],
                              sum_h.at[c].at[pl.ds(base, ZROWS)],
                              sem_sc).wait()
        pltpu.make_async_copy(deg_v, deg_h.at[wid], sem_r0).wait()

    return k(nodes, send_p, recv_p)


def _tc_finish_body(ps_ref, pd_ref, w_ref, b_ref, out_ref):
    agg = ps_ref[0] + ps_ref[1]
    deg = jnp.maximum(jnp.sum(pd_ref[...], axis=0), 1.0)
    x = agg / deg[:, None]
    out_ref[...] = (
        jnp.dot(x, w_ref[...], preferred_element_type=jnp.float32)
        + b_ref[...]
    )


def _tc_finish(part_sum, part_deg, W, b):
    R = 2048
    grid = (pl.cdiv(N_NODES, R),)
    return pl.pallas_call(
        _tc_finish_body,
        grid=grid,
        in_specs=[
            pl.BlockSpec((NC, R, D), lambda i: (0, i, 0)),
            pl.BlockSpec((NW, R), lambda i: (0, i)),
            pl.BlockSpec((D, D), lambda i: (0, 0)),
            pl.BlockSpec((1, D), lambda i: (0, 0)),
        ],
        out_specs=pl.BlockSpec((R, D), lambda i: (i, 0)),
        out_shape=jax.ShapeDtypeStruct((N_NODES, D), jnp.float32),
    )(part_sum, part_deg, W, b.reshape(1, D))


@jax.jit
def kernel(nodes, senders, receivers, W, b):
    senders = senders.astype(jnp.int32)
    receivers = receivers.astype(jnp.int32)
    pad = E_PAD - N_EDGES
    pad_iota = jnp.arange(pad, dtype=jnp.int32)
    send_p = jnp.concatenate(
        [senders, pad_iota % 128]).reshape(NW, NCHUNK, CH)
    recv_p = jnp.concatenate(
        [receivers, DUMP + pad_iota % N_DUMP]).reshape(NW, NCHUNK, CH)
    part_sum, part_deg = _sc_aggregate(nodes, send_p, recv_p)
    return _tc_finish(part_sum, part_deg, W, b)


# speedup vs baseline: 1.2913x; 1.0026x over previous
import functools

import jax
import jax.numpy as jnp
from jax import lax
from jax.experimental import pallas as pl
from jax.experimental.pallas import tpu as pltpu
from jax.experimental.pallas import tpu_sc as plsc

N_NODES = 10000
N_EDGES = 320000
D = 128
L = 16

NC = 2
NS = 16
NW = NC * NS
CH = 64
NCHUNK = 160
GRP = 8
NGRP = NCHUNK // GRP
E_PAD = NW * NCHUNK * CH
DUMP = N_NODES
ACC_ROWS = 10112
ZROWS = ACC_ROWS // NS
DEG_ROWS = 10112
N_DUMP = ACC_ROWS - N_NODES


def _sc_aggregate(nodes, send_p, recv_p):
    mesh = plsc.VectorSubcoreMesh(core_axis_name="c", subcore_axis_name="s")

    @functools.partial(
        pl.kernel,
        mesh=mesh,
        out_type=(
            jax.ShapeDtypeStruct((NC, ACC_ROWS, D), jnp.float32),
            jax.ShapeDtypeStruct((NW, DEG_ROWS), jnp.float32),
        ),
        scratch_types=[
            pltpu.VMEM_SHARED((ACC_ROWS, D), jnp.float32),
            pltpu.VMEM((2, GRP, CH), jnp.int32),
            pltpu.VMEM((2, GRP, CH), jnp.int32),
            pltpu.VMEM((4, CH, D), jnp.float32),
            pltpu.VMEM((DEG_ROWS,), jnp.float32),
            pltpu.SemaphoreType.DMA,
            pltpu.SemaphoreType.DMA,
            pltpu.SemaphoreType.DMA,
            pltpu.SemaphoreType.DMA,
            pltpu.SemaphoreType.DMA,
            pltpu.SemaphoreType.DMA,
        ],
        compiler_params=pltpu.CompilerParams(needs_layout_passes=False),
    )
    def k(nodes_h, send_h, recv_h, sum_h, deg_h, acc_sh,
          send_v, recv_v, rows_v, deg_v,
          sem_r0, sem_r1, sem_r2, sem_r3, sem_idx, sem_sc):
        c = lax.axis_index("c")
        s = lax.axis_index("s")
        wid = c * NS + s
        sems = (sem_r0, sem_r1, sem_r2, sem_r3)

        zv = jnp.zeros((L,), jnp.float32)
        ov = jnp.ones((L,), jnp.float32)

        pltpu.sync_copy(send_h.at[wid].at[pl.ds(0, GRP)], send_v.at[0])
        pltpu.sync_copy(recv_h.at[wid].at[pl.ds(0, GRP)], recv_v.at[0])
        pltpu.async_copy(
            nodes_h.at[send_v.at[0].at[0]], rows_v.at[0], sem_r0)
        pltpu.async_copy(
            nodes_h.at[send_v.at[0].at[1]], rows_v.at[1], sem_r1)
        pltpu.async_copy(
            nodes_h.at[send_v.at[0].at[2]], rows_v.at[2], sem_r2)

        @pl.loop(0, CH, unroll=8)
        def _(i):
            for j in range(D // L):
                rows_v[3, i, pl.ds(j * L, L)] = zv

        @pl.loop(0, DEG_ROWS // L, unroll=8)
        def _(i):
            deg_v[pl.ds(i * L, L)] = zv

        base = s * ZROWS
        for o in range(0, ZROWS, CH):
            n = min(CH, ZROWS - o)
            pltpu.async_copy(rows_v.at[3].at[pl.ds(0, n)],
                             acc_sh.at[pl.ds(base + o, n)], sem_sc)
        for o in range(0, ZROWS, CH):
            n = min(CH, ZROWS - o)
            pltpu.make_async_copy(rows_v.at[3].at[pl.ds(0, n)],
                                  acc_sh.at[pl.ds(base + o, n)],
                                  sem_sc).wait()

        plsc.subcore_barrier()

        @pl.loop(0, NGRP)
        def _(g):
            slot_g = lax.rem(g, 2)
            slot_n = 1 - slot_g

            @pl.when(g < NGRP - 1)
            def _():
                off = (g + 1) * GRP
                pltpu.async_copy(send_h.at[wid].at[pl.ds(off, GRP)],
                                 send_v.at[slot_n], sem_idx)
                pltpu.async_copy(recv_h.at[wid].at[pl.ds(off, GRP)],
                                 recv_v.at[slot_n], sem_idx)

            for jj in range(GRP):
                rs = jj % 4
                ns = (jj + 3) % 4
                pltpu.make_async_copy(
                    nodes_h.at[send_v.at[slot_g].at[jj]],
                    rows_v.at[rs], sems[rs]).wait()
                if jj > 0:
                    pltpu.make_async_copy(
                        rows_v.at[0],
                        acc_sh.at[recv_v.at[0].at[0]],
                        sem_sc).wait()
                else:
                    @pl.when(g > 0)
                    def _():
                        pltpu.make_async_copy(
                            rows_v.at[0],
                            acc_sh.at[recv_v.at[0].at[0]],
                            sem_sc).wait()
                if jj < GRP - 3:
                    pltpu.async_copy(
                        nodes_h.at[send_v.at[slot_g].at[jj + 3]],
                        rows_v.at[ns], sems[ns])
                elif jj == GRP - 3:
                    @pl.when(g < NGRP - 1)
                    def _():
                        pltpu.make_async_copy(
                            send_h.at[wid].at[pl.ds(0, GRP)],
                            send_v.at[slot_n], sem_idx).wait()
                        pltpu.make_async_copy(
                            recv_h.at[wid].at[pl.ds(0, GRP)],
                            recv_v.at[slot_n], sem_idx).wait()
                        pltpu.async_copy(
                            nodes_h.at[send_v.at[slot_n].at[0]],
                            rows_v.at[ns], sems[ns])
                else:
                    @pl.when(g < NGRP - 1)
                    def _():
                        pltpu.async_copy(
                            nodes_h.at[send_v.at[slot_n].at[jj - (GRP - 3)]],
                            rows_v.at[ns], sems[ns])
                pltpu.async_copy(rows_v.at[rs],
                                 acc_sh.at[recv_v.at[slot_g].at[jj]],
                                 sem_sc, add=True)
                for kk in range(CH // L):
                    rv = recv_v[slot_g, jj, pl.ds(kk * L, L)]
                    plsc.addupdate_scatter(deg_v, [rv], ov)

        pltpu.make_async_copy(
            rows_v.at[0], acc_sh.at[recv_v.at[0].at[0]], sem_sc).wait()

        plsc.subcore_barrier()

        pltpu.async_copy(acc_sh.at[pl.ds(base, ZROWS)],
                         sum_h.at[c].at[pl.ds(base, ZROWS)], sem_sc)
        pltpu.async_copy(deg_v, deg_h.at[wid], sem_r0)
        pltpu.make_async_copy(acc_sh.at[pl.ds(base, ZROWS)],
                              sum_h.at[c].at[pl.ds(base, ZROWS)],
                              sem_sc).wait()
        pltpu.make_async_copy(deg_v, deg_h.at[wid], sem_r0).wait()

    return k(nodes, send_p, recv_p)


def _tc_finish_body(ps_ref, pd_ref, w_ref, b_ref, out_ref):
    agg = ps_ref[0] + ps_ref[1]
    deg = jnp.maximum(jnp.sum(pd_ref[...], axis=0), 1.0)
    x = agg / deg[:, None]
    out_ref[...] = (
        jnp.dot(x, w_ref[...], preferred_element_type=jnp.float32)
        + b_ref[...]
    )


def _tc_finish(part_sum, part_deg, W, b):
    R = 2048
    grid = (pl.cdiv(N_NODES, R),)
    return pl.pallas_call(
        _tc_finish_body,
        grid=grid,
        in_specs=[
            pl.BlockSpec((NC, R, D), lambda i: (0, i, 0)),
            pl.BlockSpec((NW, R), lambda i: (0, i)),
            pl.BlockSpec((D, D), lambda i: (0, 0)),
            pl.BlockSpec((1, D), lambda i: (0, 0)),
        ],
        out_specs=pl.BlockSpec((R, D), lambda i: (i, 0)),
        out_shape=jax.ShapeDtypeStruct((N_NODES, D), jnp.float32),
    )(part_sum, part_deg, W, b.reshape(1, D))


@jax.jit
def kernel(nodes, senders, receivers, W, b):
    senders = senders.astype(jnp.int32)
    receivers = receivers.astype(jnp.int32)
    pad = E_PAD - N_EDGES
    pad_iota = jnp.arange(pad, dtype=jnp.int32)
    send_p = jnp.concatenate(
        [senders, pad_iota % 128]).reshape(NW, NCHUNK, CH)
    recv_p = jnp.concatenate(
        [receivers, DUMP + pad_iota % N_DUMP]).reshape(NW, NCHUNK, CH)
    part_sum, part_deg = _sc_aggregate(nodes, send_p, recv_p)
    return _tc_finish(part_sum, part_deg, W, b)
